# baseline (speedup 1.0000x reference)
import jax
import jax.numpy as jnp
from jax import lax
from jax.experimental import pallas as pl
from jax.experimental.pallas import tpu as pltpu
from jax.experimental.pallas import tpu_sc as plsc

DEPTH = 1000
N_ROWS = 16384

_D_SC = 504
_D_TC = DEPTH - _D_SC

_NC = 2
_NS = 16
_NW = _NC * _NS
_COLS_PER_W = N_ROWS // _NW
_SUB = 128
_NSUB = _COLS_PER_W // _SUB
_GROUPS = _SUB // 16

_TC_BLK = 512


def _onehot_sc(idx_hbm, zeros_hbm, outT_hbm, idx_v, buf, sem, semz):
    wid = lax.axis_index("s") * _NC + lax.axis_index("c")
    col0 = wid * _COLS_PER_W

    cpz = pltpu.make_async_copy(zeros_hbm, buf, semz)
    cpz.start()
    pltpu.sync_copy(idx_hbm.at[pl.ds(col0, _COLS_PER_W)], idx_v)
    cpz.wait()

    lane = jnp.arange(16, dtype=jnp.int32)
    ones16 = jnp.ones((16,), jnp.float32)
    zeros16 = jnp.zeros((16,), jnp.float32)
    prev = None

    for c in range(_NSUB):
        if c > 0:
            prev.wait()
            for g in range(_GROUPS):
                vals = idx_v[pl.ds((c - 1) * _SUB + g * 16, 16)]
                m = vals < _D_SC
                plsc.store_scatter(buf, [vals, lane + g * 16], zeros16, mask=m)
        for g in range(_GROUPS):
            vals = idx_v[pl.ds(c * _SUB + g * 16, 16)]
            m = vals < _D_SC
            plsc.store_scatter(buf, [vals, lane + g * 16], ones16, mask=m)
        dst = outT_hbm.at[pl.ds(0, _D_SC), pl.ds(col0 + c * _SUB, _SUB)]
        prev = pltpu.make_async_copy(buf, dst, sem)
        prev.start()

    prev.wait()


def _onehot_tc(idx_ref, out_ref):
    j = pl.program_id(0)
    vals = idx_ref[pl.ds(j * _TC_BLK, _TC_BLK)]
    d = lax.broadcasted_iota(jnp.int32, (_D_TC, _TC_BLK), 0) + _D_SC
    out_ref[...] = (d == vals[None, :]).astype(jnp.float32)


@jax.jit
def _onehot(idx_flat):
    mesh = plsc.VectorSubcoreMesh(core_axis_name="c", subcore_axis_name="s")
    zeros = jnp.zeros((_D_SC, _SUB), jnp.float32)
    out_sc = pl.kernel(
        _onehot_sc,
        out_type=jax.ShapeDtypeStruct((_D_SC, N_ROWS), jnp.float32),
        mesh=mesh,
        scratch_types=[
            pltpu.VMEM((_COLS_PER_W,), jnp.int32),
            pltpu.VMEM((_D_SC, _SUB), jnp.float32),
            pltpu.SemaphoreType.DMA,
            pltpu.SemaphoreType.DMA,
        ],
        compiler_params=pltpu.CompilerParams(
            needs_layout_passes=False, use_tc_tiling_on_sc=True
        ),
    )(idx_flat, zeros)

    out_tc = pl.pallas_call(
        _onehot_tc,
        grid=(N_ROWS // _TC_BLK,),
        in_specs=[pl.BlockSpec((N_ROWS,), lambda j: (0,))],
        out_specs=pl.BlockSpec((_D_TC, _TC_BLK), lambda j: (0, j)),
        out_shape=jax.ShapeDtypeStruct((_D_TC, N_ROWS), jnp.float32),
    )(idx_flat)

    return jnp.concatenate([out_sc, out_tc], axis=0).T


def kernel(inputs):
    idx_flat = inputs.astype(jnp.int32).reshape(-1)
    return _onehot(idx_flat)


# speedup vs baseline: 1.2404x; 1.2404x over previous
import jax
import jax.numpy as jnp
from jax import lax
from jax.experimental import pallas as pl
from jax.experimental.pallas import tpu as pltpu
from jax.experimental.pallas import tpu_sc as plsc

DEPTH = 1000
N_ROWS = 16384

_NC = 2
_NS = 16
_NW = _NC * _NS
_COLS_PER_W = N_ROWS // _NW
_SUB = 128
_NSUB = _COLS_PER_W // _SUB
_GROUPS = _SUB // 16

_PART = 256
_NPART = 4
_PART_SIZES = (256, 256, 256, 232)


def _onehot_sc(idx_hbm, zeros_hbm, outT_hbm, idx_v, b0, b1, b2, b3,
               s0, s1, s2, s3):
    wid = lax.axis_index("s") * _NC + lax.axis_index("c")
    col0 = wid * _COLS_PER_W
    bufs = (b0, b1, b2, b3)
    sems = (s0, s1, s2, s3)

    zfills = []
    for p in range(_NPART):
        cp = pltpu.make_async_copy(
            zeros_hbm.at[pl.ds(0, _PART_SIZES[p])], bufs[p], sems[p]
        )
        cp.start()
        zfills.append(cp)
    pltpu.sync_copy(idx_hbm.at[pl.ds(col0, _COLS_PER_W)], idx_v)
    for cp in zfills:
        cp.wait()

    lane = jnp.arange(16, dtype=jnp.int32)
    ones16 = jnp.ones((16,), jnp.float32)
    zeros16 = jnp.zeros((16,), jnp.float32)

    def scatter_subchunk(c, x16):
        for g in range(_GROUPS):
            vals = idx_v[pl.ds(c * _SUB + g * 16, 16)]
            part = lax.shift_right_logical(vals, 8)
            row = vals - part * _PART
            col = lane + g * 16
            for p in range(_NPART):
                plsc.store_scatter(
                    bufs[p], [row, col], x16, mask=part == p
                )

    copies = [None] * _NPART
    for c in range(_NSUB):
        if c > 0:
            for p in range(_NPART):
                copies[p].wait()
            scatter_subchunk(c - 1, zeros16)
        scatter_subchunk(c, ones16)
        for p in range(_NPART):
            dst = outT_hbm.at[
                pl.ds(p * _PART, _PART_SIZES[p]),
                pl.ds(col0 + c * _SUB, _SUB),
            ]
            copies[p] = pltpu.make_async_copy(bufs[p], dst, sems[p])
            copies[p].start()

    for p in range(_NPART):
        copies[p].wait()


@jax.jit
def _onehot(idx_flat):
    mesh = plsc.VectorSubcoreMesh(core_axis_name="c", subcore_axis_name="s")
    zeros = jnp.zeros((_PART, _SUB), jnp.float32)
    out_t = pl.kernel(
        _onehot_sc,
        out_type=jax.ShapeDtypeStruct((DEPTH, N_ROWS), jnp.float32),
        mesh=mesh,
        scratch_types=[
            pltpu.VMEM((_COLS_PER_W,), jnp.int32),
            pltpu.VMEM((_PART_SIZES[0], _SUB), jnp.float32),
            pltpu.VMEM((_PART_SIZES[1], _SUB), jnp.float32),
            pltpu.VMEM((_PART_SIZES[2], _SUB), jnp.float32),
            pltpu.VMEM((_PART_SIZES[3], _SUB), jnp.float32),
            pltpu.SemaphoreType.DMA,
            pltpu.SemaphoreType.DMA,
            pltpu.SemaphoreType.DMA,
            pltpu.SemaphoreType.DMA,
        ],
        compiler_params=pltpu.CompilerParams(
            needs_layout_passes=False, use_tc_tiling_on_sc=True
        ),
    )(idx_flat, zeros)
    return out_t.T


def kernel(inputs):
    idx_flat = inputs.astype(jnp.int32).reshape(-1)
    return _onehot(idx_flat)


# speedup vs baseline: 1.6676x; 1.3444x over previous
import jax
import jax.numpy as jnp
import numpy as np
from jax import lax
from jax.experimental import pallas as pl
from jax.experimental.pallas import tpu as pltpu
from jax.experimental.pallas import tpu_sc as plsc

DEPTH = 1000
N_ROWS = 16384

_NC = 2
_NS = 16
_NW = _NC * _NS
_COLS_PER_W = N_ROWS // _NW
_SUB = 128
_NSUB = _COLS_PER_W // _SUB
_GROUPS = _SUB // 16


def _onehot_sc(idx_hbm, zeros_hbm, outT_hbm, idx_v, buf, sem, semz):
    wid = lax.axis_index("s") * _NC + lax.axis_index("c")
    col0 = wid * _COLS_PER_W

    cpz = pltpu.make_async_copy(zeros_hbm, buf, semz)
    cpz.start()
    pltpu.sync_copy(idx_hbm.at[pl.ds(col0, _COLS_PER_W)], idx_v)
    cpz.wait()

    lane = jnp.arange(16, dtype=jnp.int32)
    ones16 = jnp.ones((16,), jnp.float32)
    zeros16 = jnp.zeros((16,), jnp.float32)
    prev = None

    for c in range(_NSUB):
        if c > 0:
            prev.wait()
            for g in range(_GROUPS):
                vals = idx_v[pl.ds((c - 1) * _SUB + g * 16, 16)]
                plsc.store_scatter(buf, [vals, lane + g * 16], zeros16)
        for g in range(_GROUPS):
            vals = idx_v[pl.ds(c * _SUB + g * 16, 16)]
            plsc.store_scatter(buf, [vals, lane + g * 16], ones16)
        dst = outT_hbm.at[pl.ds(0, DEPTH), pl.ds(col0 + c * _SUB, _SUB)]
        prev = pltpu.make_async_copy(buf, dst, sem)
        prev.start()

    prev.wait()


@jax.jit
def _onehot(idx_flat):
    mesh = plsc.VectorSubcoreMesh(core_axis_name="c", subcore_axis_name="s")
    zeros = np.zeros((DEPTH, _SUB), np.float32)
    out_t = pl.kernel(
        _onehot_sc,
        out_type=jax.ShapeDtypeStruct((DEPTH, N_ROWS), jnp.float32),
        mesh=mesh,
        scratch_types=[
            pltpu.VMEM((_COLS_PER_W,), jnp.int32),
            pltpu.VMEM((DEPTH, _SUB), jnp.float32),
            pltpu.SemaphoreType.DMA,
            pltpu.SemaphoreType.DMA,
        ],
        compiler_params=pltpu.CompilerParams(
            needs_layout_passes=False, use_tc_tiling_on_sc=True
        ),
    )(idx_flat, zeros)
    return out_t.T


def kernel(inputs):
    idx_flat = inputs.astype(jnp.int32).reshape(-1)
    return _onehot(idx_flat)
